# Initial kernel scaffold; baseline (speedup 1.0000x reference)
import jax
import jax.numpy as jnp
from jax.experimental import pallas as pl


def kernel(fine_ids, coarse_ids, fine_table, coarse_table, W1, b1, W2, b2):
    raise NotImplementedError("write your pallas kernel here")


# speedup vs baseline: 3.1629x; 3.1629x over previous
import jax
import jax.numpy as jnp
from jax import lax
from jax.experimental import pallas as pl
from jax.experimental.pallas import tpu as pltpu
from jax.experimental.pallas import tpu_sc as plsc

B, L, DIM = 4096, 50, 64
N = B * L
NC, NS = 2, 16
NW = NC * NS
PER_W = N // NW
CH = 128
NCH = PER_W // CH

def _sc_gather_body(fidx_hbm, cidx_hbm, ftab_hbm, ctab_hbm,
                    fout_hbm, cout_hbm,
                    fidx_v, cidx_v, fbuf, cbuf, fsem, csem):
    wid = lax.axis_index("s") * NC + lax.axis_index("c")
    base = wid * PER_W
    pltpu.sync_copy(fidx_hbm.at[wid], fidx_v)
    pltpu.sync_copy(cidx_hbm.at[wid], cidx_v)

    def step(j, carry):
        cp_f = pltpu.async_copy(ftab_hbm.at[fidx_v.at[j]], fbuf, fsem)
        cp_c = pltpu.async_copy(ctab_hbm.at[cidx_v.at[j]], cbuf, csem)
        cp_f.wait()
        cp_c.wait()
        pltpu.sync_copy(fbuf, fout_hbm.at[pl.ds(base + j * CH, CH)])
        pltpu.sync_copy(cbuf, cout_hbm.at[pl.ds(base + j * CH, CH)])
        return carry

    lax.fori_loop(0, NCH, step, 0)


def _sc_gather(fidx, cidx, ftab, ctab):
    return pl.kernel(
        _sc_gather_body,
        out_type=(
            jax.ShapeDtypeStruct((N, DIM), jnp.float32),
            jax.ShapeDtypeStruct((N, DIM), jnp.float32),
        ),
        mesh=plsc.VectorSubcoreMesh(core_axis_name="c", subcore_axis_name="s",
                                    num_cores=NC, num_subcores=NS),
        scratch_types=[
            pltpu.VMEM((NCH, CH), jnp.int32),
            pltpu.VMEM((NCH, CH), jnp.int32),
            pltpu.VMEM((CH, DIM), jnp.float32),
            pltpu.VMEM((CH, DIM), jnp.float32),
            pltpu.SemaphoreType.DMA,
            pltpu.SemaphoreType.DMA,
        ],
        compiler_params=pltpu.CompilerParams(use_tc_tiling_on_sc=False),
    )(fidx, cidx, ftab, ctab)

BLK = 2048


def _tc_mlp_body(f_ref, c_ref, w1a_ref, w1b_ref, b1_ref, w2_ref, b2_ref,
                 o_ref):
    h = jnp.dot(f_ref[...], w1a_ref[...], preferred_element_type=jnp.float32)
    h = h + jnp.dot(c_ref[...], w1b_ref[...],
                    preferred_element_type=jnp.float32)
    h = jnp.maximum(h + b1_ref[...], 0.0)
    o_ref[...] = (jnp.dot(h, w2_ref[...], preferred_element_type=jnp.float32)
                  + b2_ref[...])


def _tc_mlp(frows, crows, w1a, w1b, b1, w2, b2, *, interpret=False):
    return pl.pallas_call(
        _tc_mlp_body,
        grid=(N // BLK,),
        in_specs=[
            pl.BlockSpec((BLK, DIM), lambda i: (i, 0)),
            pl.BlockSpec((BLK, DIM), lambda i: (i, 0)),
            pl.BlockSpec((DIM, 2 * DIM), lambda i: (0, 0)),
            pl.BlockSpec((DIM, 2 * DIM), lambda i: (0, 0)),
            pl.BlockSpec((1, 2 * DIM), lambda i: (0, 0)),
            pl.BlockSpec((2 * DIM, DIM), lambda i: (0, 0)),
            pl.BlockSpec((1, DIM), lambda i: (0, 0)),
        ],
        out_specs=pl.BlockSpec((BLK, DIM), lambda i: (i, 0)),
        out_shape=jax.ShapeDtypeStruct((N, DIM), jnp.float32),
        interpret=interpret,
    )(frows, crows, w1a, w1b, b1, w2, b2)


def kernel(fine_ids, coarse_ids, fine_table, coarse_table, W1, b1, W2, b2):
    fidx = fine_ids.reshape(NW, NCH, CH).astype(jnp.int32)
    cidx = coarse_ids.reshape(NW, NCH, CH).astype(jnp.int32)
    frows, crows = _sc_gather(fidx, cidx, fine_table, coarse_table)
    out = _tc_mlp(frows, crows, W1[:DIM], W1[DIM:], b1.reshape(1, 2 * DIM),
                  W2, b2.reshape(1, DIM))
    return out.reshape(B, L, DIM), jnp.float32(0.5)


# speedup vs baseline: 4.6431x; 1.4680x over previous
import jax
import jax.numpy as jnp
from jax import lax
from jax.experimental import pallas as pl
from jax.experimental.pallas import tpu as pltpu
from jax.experimental.pallas import tpu_sc as plsc

B, L, DIM = 4096, 50, 64
N = B * L
NC, NS = 2, 16
NW = NC * NS
PER_W = N // NW
CH = 128
NCH = PER_W // CH


def _sc_gather_body(fidx_hbm, cidx_hbm, ftab_hbm, ctab_hbm,
                    fout_hbm, cout_hbm,
                    fidx_v, cidx_v, fbuf, cbuf, fsem, csem):
    wid = lax.axis_index("s") * NC + lax.axis_index("c")
    base = wid * PER_W
    pltpu.sync_copy(fidx_hbm.at[pl.ds(base, PER_W)], fidx_v)
    pltpu.sync_copy(cidx_hbm.at[pl.ds(base, PER_W)], cidx_v)

    def step(j, carry):
        idx_f = fidx_v.at[pl.ds(j * CH, CH)]
        idx_c = cidx_v.at[pl.ds(j * CH, CH)]
        cp_f = pltpu.async_copy(ftab_hbm.at[idx_f], fbuf, fsem)
        cp_c = pltpu.async_copy(ctab_hbm.at[idx_c], cbuf, csem)
        cp_f.wait()
        cp_c.wait()
        pltpu.sync_copy(fbuf, fout_hbm.at[pl.ds(base + j * CH, CH)])
        pltpu.sync_copy(cbuf, cout_hbm.at[pl.ds(base + j * CH, CH)])
        return carry

    lax.fori_loop(0, NCH, step, 0)


def _sc_gather(fidx, cidx, ftab, ctab):
    return pl.kernel(
        _sc_gather_body,
        out_type=(
            jax.ShapeDtypeStruct((N, DIM), jnp.float32),
            jax.ShapeDtypeStruct((N, DIM), jnp.float32),
        ),
        mesh=plsc.VectorSubcoreMesh(core_axis_name="c", subcore_axis_name="s",
                                    num_cores=NC, num_subcores=NS),
        scratch_types=[
            pltpu.VMEM((PER_W,), jnp.int32),
            pltpu.VMEM((PER_W,), jnp.int32),
            pltpu.VMEM((CH, DIM), jnp.float32),
            pltpu.VMEM((CH, DIM), jnp.float32),
            pltpu.SemaphoreType.DMA,
            pltpu.SemaphoreType.DMA,
        ],
        compiler_params=pltpu.CompilerParams(use_tc_tiling_on_sc=False),
    )(fidx, cidx, ftab, ctab)


BLK = 1024
N2 = N // 2


def _tc_mlp_body(f_ref, c_ref, w1a_ref, w1b_ref, b1_ref, w2_ref, b2_ref,
                 o_ref):
    h = jnp.dot(f_ref[...], w1a_ref[...], preferred_element_type=jnp.float32)
    h = h + jnp.dot(c_ref[...], w1b_ref[...],
                    preferred_element_type=jnp.float32)
    h = jnp.maximum(h + b1_ref[...], 0.0)
    o_ref[...] = (jnp.dot(h, w2_ref[...], preferred_element_type=jnp.float32)
                  + b2_ref[...])


def _tc_mlp(f2, c2, w1a2, w1b2, b1_2, w2_2, b2_2, *, interpret=False):
    return pl.pallas_call(
        _tc_mlp_body,
        grid=(N2 // BLK,),
        in_specs=[
            pl.BlockSpec((BLK, 2 * DIM), lambda i: (i, 0)),
            pl.BlockSpec((BLK, 2 * DIM), lambda i: (i, 0)),
            pl.BlockSpec((2 * DIM, 4 * DIM), lambda i: (0, 0)),
            pl.BlockSpec((2 * DIM, 4 * DIM), lambda i: (0, 0)),
            pl.BlockSpec((1, 4 * DIM), lambda i: (0, 0)),
            pl.BlockSpec((4 * DIM, 2 * DIM), lambda i: (0, 0)),
            pl.BlockSpec((1, 2 * DIM), lambda i: (0, 0)),
        ],
        out_specs=pl.BlockSpec((BLK, 2 * DIM), lambda i: (i, 0)),
        out_shape=jax.ShapeDtypeStruct((N2, 2 * DIM), jnp.float32),
        interpret=interpret,
    )(f2, c2, w1a2, w1b2, b1_2, w2_2, b2_2)


def _paired_weights(W1, b1, W2, b2):
    z = jnp.zeros((DIM, 2 * DIM), jnp.float32)
    w1a = W1[:DIM]
    w1b = W1[DIM:]
    w1a2 = jnp.block([[w1a, z], [z, w1a]])
    w1b2 = jnp.block([[w1b, z], [z, w1b]])
    z2 = jnp.zeros((2 * DIM, DIM), jnp.float32)
    w2_2 = jnp.block([[W2, z2], [z2, W2]])
    b1_2 = jnp.concatenate([b1, b1]).reshape(1, 4 * DIM)
    b2_2 = jnp.concatenate([b2, b2]).reshape(1, 2 * DIM)
    return w1a2, w1b2, b1_2, w2_2, b2_2


def kernel(fine_ids, coarse_ids, fine_table, coarse_table, W1, b1, W2, b2):
    fidx = fine_ids.reshape(N).astype(jnp.int32)
    cidx = coarse_ids.reshape(N).astype(jnp.int32)
    frows, crows = _sc_gather(fidx, cidx, fine_table, coarse_table)
    f2 = frows.reshape(N2, 2 * DIM)
    c2 = crows.reshape(N2, 2 * DIM)
    w1a2, w1b2, b1_2, w2_2, b2_2 = _paired_weights(W1, b1, W2, b2)
    out2 = _tc_mlp(f2, c2, w1a2, w1b2, b1_2, w2_2, b2_2)
    return out2.reshape(B, L, DIM), jnp.float32(0.5)


# speedup vs baseline: 4.6444x; 1.0003x over previous
import jax
import jax.numpy as jnp
from jax import lax
from jax.experimental import pallas as pl
from jax.experimental.pallas import tpu as pltpu
from jax.experimental.pallas import tpu_sc as plsc

B, L, DIM = 4096, 50, 64
N = B * L
NC, NS = 2, 16
NW = NC * NS
PER_W = N // NW
CH = 128
NCH = PER_W // CH


def _sc_gather_body(fidx_hbm, cidx_hbm, ftab_hbm, ctab_hbm,
                    fout_hbm, cout_hbm,
                    fidx_v, cidx_v, fbuf, cbuf, fsems, csems):
    wid = lax.axis_index("s") * NC + lax.axis_index("c")
    base = wid * PER_W
    pltpu.sync_copy(fidx_hbm.at[pl.ds(base, PER_W)], fidx_v)
    pltpu.sync_copy(cidx_hbm.at[pl.ds(base, PER_W)], cidx_v)

    def gather(j, slot):
        jc = lax.min(j, NCH - 1)
        pltpu.async_copy(ftab_hbm.at[fidx_v.at[pl.ds(jc * CH, CH)]],
                         fbuf.at[slot], fsems.at[slot])
        pltpu.async_copy(ctab_hbm.at[cidx_v.at[pl.ds(jc * CH, CH)]],
                         cbuf.at[slot], csems.at[slot])

    def wait_write(j, slot):
        pltpu.make_async_copy(ftab_hbm.at[fidx_v.at[pl.ds(0, CH)]],
                              fbuf.at[slot], fsems.at[slot]).wait()
        pltpu.make_async_copy(ctab_hbm.at[cidx_v.at[pl.ds(0, CH)]],
                              cbuf.at[slot], csems.at[slot]).wait()
        pltpu.sync_copy(fbuf.at[slot], fout_hbm.at[pl.ds(base + j * CH, CH)])
        pltpu.sync_copy(cbuf.at[slot], cout_hbm.at[pl.ds(base + j * CH, CH)])

    gather(0, 0)
    gather(1, 1)

    def pair(jj, carry):
        j0 = jj * 2
        wait_write(j0, 0)
        gather(j0 + 2, 0)
        wait_write(j0 + 1, 1)
        gather(j0 + 3, 1)
        return carry

    lax.fori_loop(0, NCH // 2, pair, 0)
    for slot in (0, 1):
        pltpu.make_async_copy(ftab_hbm.at[fidx_v.at[pl.ds(0, CH)]],
                              fbuf.at[slot], fsems.at[slot]).wait()
        pltpu.make_async_copy(ctab_hbm.at[cidx_v.at[pl.ds(0, CH)]],
                              cbuf.at[slot], csems.at[slot]).wait()


def _sc_gather(fidx, cidx, ftab, ctab):
    return pl.kernel(
        _sc_gather_body,
        out_type=(
            jax.ShapeDtypeStruct((N, DIM), jnp.float32),
            jax.ShapeDtypeStruct((N, DIM), jnp.float32),
        ),
        mesh=plsc.VectorSubcoreMesh(core_axis_name="c", subcore_axis_name="s",
                                    num_cores=NC, num_subcores=NS),
        scratch_types=[
            pltpu.VMEM((PER_W,), jnp.int32),
            pltpu.VMEM((PER_W,), jnp.int32),
            pltpu.VMEM((2, CH, DIM), jnp.float32),
            pltpu.VMEM((2, CH, DIM), jnp.float32),
            pltpu.SemaphoreType.DMA((2,)),
            pltpu.SemaphoreType.DMA((2,)),
        ],
        compiler_params=pltpu.CompilerParams(use_tc_tiling_on_sc=False),
    )(fidx, cidx, ftab, ctab)


BLK = 1024
N2 = N // 2


def _tc_mlp_body(f_ref, c_ref, w1a_ref, w1b_ref, b1_ref, w2_ref, b2_ref,
                 o_ref):
    h = jnp.dot(f_ref[...], w1a_ref[...], preferred_element_type=jnp.float32)
    h = h + jnp.dot(c_ref[...], w1b_ref[...],
                    preferred_element_type=jnp.float32)
    h = jnp.maximum(h + b1_ref[...], 0.0)
    o_ref[...] = (jnp.dot(h, w2_ref[...], preferred_element_type=jnp.float32)
                  + b2_ref[...])


def _tc_mlp(f2, c2, w1a2, w1b2, b1_2, w2_2, b2_2, *, interpret=False):
    return pl.pallas_call(
        _tc_mlp_body,
        grid=(N2 // BLK,),
        in_specs=[
            pl.BlockSpec((BLK, 2 * DIM), lambda i: (i, 0)),
            pl.BlockSpec((BLK, 2 * DIM), lambda i: (i, 0)),
            pl.BlockSpec((2 * DIM, 4 * DIM), lambda i: (0, 0)),
            pl.BlockSpec((2 * DIM, 4 * DIM), lambda i: (0, 0)),
            pl.BlockSpec((1, 4 * DIM), lambda i: (0, 0)),
            pl.BlockSpec((4 * DIM, 2 * DIM), lambda i: (0, 0)),
            pl.BlockSpec((1, 2 * DIM), lambda i: (0, 0)),
        ],
        out_specs=pl.BlockSpec((BLK, 2 * DIM), lambda i: (i, 0)),
        out_shape=jax.ShapeDtypeStruct((N2, 2 * DIM), jnp.float32),
        interpret=interpret,
    )(f2, c2, w1a2, w1b2, b1_2, w2_2, b2_2)


def _paired_weights(W1, b1, W2, b2):
    z = jnp.zeros((DIM, 2 * DIM), jnp.float32)
    w1a = W1[:DIM]
    w1b = W1[DIM:]
    w1a2 = jnp.block([[w1a, z], [z, w1a]])
    w1b2 = jnp.block([[w1b, z], [z, w1b]])
    z2 = jnp.zeros((2 * DIM, DIM), jnp.float32)
    w2_2 = jnp.block([[W2, z2], [z2, W2]])
    b1_2 = jnp.concatenate([b1, b1]).reshape(1, 4 * DIM)
    b2_2 = jnp.concatenate([b2, b2]).reshape(1, 2 * DIM)
    return w1a2, w1b2, b1_2, w2_2, b2_2


def kernel(fine_ids, coarse_ids, fine_table, coarse_table, W1, b1, W2, b2):
    fidx = fine_ids.reshape(N).astype(jnp.int32)
    cidx = coarse_ids.reshape(N).astype(jnp.int32)
    frows, crows = _sc_gather(fidx, cidx, fine_table, coarse_table)
    f2 = frows.reshape(N2, 2 * DIM)
    c2 = crows.reshape(N2, 2 * DIM)
    w1a2, w1b2, b1_2, w2_2, b2_2 = _paired_weights(W1, b1, W2, b2)
    out2 = _tc_mlp(f2, c2, w1a2, w1b2, b1_2, w2_2, b2_2)
    return out2.reshape(B, L, DIM), jnp.float32(0.5)
